# baseline (speedup 1.0000x reference)
import jax
import jax.numpy as jnp
from jax import lax
from jax.experimental import pallas as pl
from jax.experimental.pallas import tpu as pltpu
from jax.experimental.pallas import tpu_sc as plsc

_N = 10000
_E = 320000
_D = 128
_H = 8
_C = 16
_OUT = 64
_PW = 144
_NC = 2
_NS = 16
_NW = _NC * _NS
_EPW = _E // _NW
_B = 80
_CPW = _EPW // _B
_SEG = 25
_NSEG = _CPW // _SEG
_NPAD = 10240
_RPT = _NPAD // _NS
_RZ = 16
_BLK = 2000


def _dense_body(x_ref, w_ref, asd_ref, h_ref, ts_ref, td_ref):
    h = jnp.dot(x_ref[...], w_ref[...], preferred_element_type=jnp.float32)
    h_ref[...] = h
    sal = jnp.dot(h, asd_ref[...], preferred_element_type=jnp.float32)
    ts_ref[...] = jnp.concatenate([sal[:, :_H], sal[:, :_H]], axis=1)
    td_ref[...] = jnp.concatenate([sal[:, _H:], sal[:, _H:]], axis=1)


def _dense(x, W, ASD):
    return pl.pallas_call(
        _dense_body,
        grid=(_N // _BLK,),
        in_specs=[
            pl.BlockSpec((_BLK, _D), lambda i: (i, 0)),
            pl.BlockSpec((_D, _D), lambda i: (0, 0)),
            pl.BlockSpec((_D, 2 * _H), lambda i: (0, 0)),
        ],
        out_specs=[
            pl.BlockSpec((_BLK, _D), lambda i: (i, 0)),
            pl.BlockSpec((_BLK, 2 * _H), lambda i: (i, 0)),
            pl.BlockSpec((_BLK, 2 * _H), lambda i: (i, 0)),
        ],
        out_shape=[
            jax.ShapeDtypeStruct((_N, _D), jnp.float32),
            jax.ShapeDtypeStruct((_N, 2 * _H), jnp.float32),
            jax.ShapeDtypeStruct((_N, 2 * _H), jnp.float32),
        ],
    )(x, W, ASD)


def _combine_common(acc_ref, perm_ref, h_ref, ts_ref, td_ref, b_ref, p8_ref):
    a0 = acc_ref[0]
    a1 = acc_ref[1]
    nump = a0[:, :_D] + a1[:, :_D]
    num = jnp.dot(nump, perm_ref[...], preferred_element_type=jnp.float32)
    den8 = a0[:, _D:_D + _H] + a1[:, _D:_D + _H]
    es = ts_ref[:, :_H] + td_ref[:, :_H]
    ws8 = jnp.exp(jnp.maximum(es, 0.2 * es))
    p8 = p8_ref[...]
    wsbig = jnp.dot(ws8, p8, preferred_element_type=jnp.float32)
    denbig = jnp.dot(den8 + ws8, p8, preferred_element_type=jnp.float32)
    v = (num + wsbig * h_ref[...]) / denbig + b_ref[...]
    return jnp.where(v > 0, v, jnp.exp(jnp.minimum(v, 0.0)) - 1.0)


def _combine_mid_body(acc_ref, perm_ref, h_ref, ts_ref, td_ref, b_ref, p8_ref,
                      w_ref, asd_ref, h2_ref, ts2_ref, td2_ref):
    x2 = _combine_common(acc_ref, perm_ref, h_ref, ts_ref, td_ref, b_ref,
                         p8_ref)
    h2 = jnp.dot(x2, w_ref[...], preferred_element_type=jnp.float32)
    h2_ref[...] = h2
    sal2 = jnp.dot(h2, asd_ref[...], preferred_element_type=jnp.float32)
    ts2_ref[...] = jnp.concatenate([sal2[:, :_H], sal2[:, :_H]], axis=1)
    td2_ref[...] = jnp.concatenate([sal2[:, _H:], sal2[:, _H:]], axis=1)


def _combine_mid(acc, PERM, h, ts, td, b2d, P8, W, ASD):
    return pl.pallas_call(
        _combine_mid_body,
        grid=(_N // _BLK,),
        in_specs=[
            pl.BlockSpec((_NC, _BLK, _PW), lambda i: (0, i, 0)),
            pl.BlockSpec((_D, _D), lambda i: (0, 0)),
            pl.BlockSpec((_BLK, _D), lambda i: (i, 0)),
            pl.BlockSpec((_BLK, 2 * _H), lambda i: (i, 0)),
            pl.BlockSpec((_BLK, 2 * _H), lambda i: (i, 0)),
            pl.BlockSpec((1, _D), lambda i: (0, 0)),
            pl.BlockSpec((_H, _D), lambda i: (0, 0)),
            pl.BlockSpec((_D, _D), lambda i: (0, 0)),
            pl.BlockSpec((_D, 2 * _H), lambda i: (0, 0)),
        ],
        out_specs=[
            pl.BlockSpec((_BLK, _D), lambda i: (i, 0)),
            pl.BlockSpec((_BLK, 2 * _H), lambda i: (i, 0)),
            pl.BlockSpec((_BLK, 2 * _H), lambda i: (i, 0)),
        ],
        out_shape=[
            jax.ShapeDtypeStruct((_N, _D), jnp.float32),
            jax.ShapeDtypeStruct((_N, 2 * _H), jnp.float32),
            jax.ShapeDtypeStruct((_N, 2 * _H), jnp.float32),
        ],
    )(acc, PERM, h, ts, td, b2d, P8, W, ASD)


def _final_body(acc_ref, perm_ref, h_ref, ts_ref, td_ref, b_ref, p8_ref,
                wout_ref, bout_ref, out_ref):
    x2 = _combine_common(acc_ref, perm_ref, h_ref, ts_ref, td_ref, b_ref,
                         p8_ref)
    z = jnp.dot(x2, wout_ref[...], preferred_element_type=jnp.float32)
    z = z + bout_ref[...]
    out_ref[...] = 1.0 / (1.0 + jnp.exp(-z))


def _final(acc, PERM, h, ts, td, b2d, P8, Wout, bout2d):
    return pl.pallas_call(
        _final_body,
        grid=(_N // _BLK,),
        in_specs=[
            pl.BlockSpec((_NC, _BLK, _PW), lambda i: (0, i, 0)),
            pl.BlockSpec((_D, _D), lambda i: (0, 0)),
            pl.BlockSpec((_BLK, _D), lambda i: (i, 0)),
            pl.BlockSpec((_BLK, 2 * _H), lambda i: (i, 0)),
            pl.BlockSpec((_BLK, 2 * _H), lambda i: (i, 0)),
            pl.BlockSpec((1, _D), lambda i: (0, 0)),
            pl.BlockSpec((_H, _D), lambda i: (0, 0)),
            pl.BlockSpec((_D, _OUT), lambda i: (0, 0)),
            pl.BlockSpec((1, _OUT), lambda i: (0, 0)),
        ],
        out_specs=pl.BlockSpec((_BLK, _OUT), lambda i: (i, 0)),
        out_shape=jax.ShapeDtypeStruct((_N, _OUT), jnp.float32),
    )(acc, PERM, h, ts, td, b2d, P8, Wout, bout2d)


def _edge_body(hpk_hbm, ts_hbm, td_hbm, ei2d_hbm, out_hbm,
               sseg_v, dseg_v,
               sal_s0, sal_d0, hpk0,
               sal_s1, sal_d1, hpk1,
               pay_v, zbuf_v, acc_sh,
               gsem0, gsem1, ssem):
    cid = lax.axis_index("c")
    sid = lax.axis_index("s")
    wid = sid * _NC + cid
    tile_row0 = sid * _RPT

    bufs = ((sal_s0, sal_d0, hpk0, gsem0),
            (sal_s1, sal_d1, hpk1, gsem1))

    for r in range(_RZ):
        for j in range(_PW // 16):
            zbuf_v[r, pl.ds(j * 16, 16)] = jnp.zeros((16,), jnp.float32)

    def zero_body(k, carry):
        pltpu.sync_copy(zbuf_v, acc_sh.at[pl.ds(tile_row0 + k * _RZ, _RZ)])
        return carry

    lax.fori_loop(0, _RPT // _RZ, zero_body, 0)
    plsc.subcore_barrier()

    iota16 = lax.iota(jnp.int32, 16)
    half = iota16 // 8

    def gather_copies(k, b):
        sal_s, sal_d, hpk, gsem = bufs[b]
        sidx = sseg_v.at[k]
        didx = dseg_v.at[k]
        return (pltpu.make_async_copy(ts_hbm.at[sidx], sal_s, gsem),
                pltpu.make_async_copy(td_hbm.at[didx], sal_d, gsem),
                pltpu.make_async_copy(hpk_hbm.at[sidx], hpk, gsem))

    def issue_gathers(k, b):
        for c in gather_copies(k, b):
            c.start()

    def wait_gathers(k, b):
        for c in gather_copies(k, b):
            c.wait()

    def scatter_issue(k):
        pltpu.async_copy(pay_v, acc_sh.at[dseg_v.at[k]], ssem, add=True)

    def scatter_wait(k):
        pltpu.make_async_copy(pay_v, acc_sh.at[dseg_v.at[k]], ssem).wait()

    def compute(b):
        sal_s, sal_d, hpk, _ = bufs[b]

        @plsc.parallel_loop(0, _B, unroll=2)
        def _(i):
            e = sal_s[i, :] + sal_d[i, :]
            e = jnp.maximum(e, 0.2 * e)
            w = jnp.exp(e)
            pay_v[i, pl.ds(_D, 16)] = w
            for g in range(4):
                v = hpk[i, pl.ds(g * 32, 32)]
                lo, hi = plsc.unpack(v, format=plsc.PackFormat.INTERLEAVED)
                wmix = jnp.take_along_axis(w, 2 * g + half, axis=0)
                pay_v[i, pl.ds(g * 32, 16)] = lo * wmix
                pay_v[i, pl.ds(g * 32 + 16, 16)] = hi * wmix

    for seg in range(_NSEG):
        row0 = wid * _CPW + seg * _SEG
        pltpu.sync_copy(ei2d_hbm.at[pl.ds(row0, _SEG)], sseg_v)
        pltpu.sync_copy(ei2d_hbm.at[pl.ds(_E // _B + row0, _SEG)], dseg_v)

        issue_gathers(0, 0)

        def pair_body(g, carry):
            k0 = 2 * g
            issue_gathers(k0 + 1, 1)
            wait_gathers(k0, 0)

            @pl.when(g > 0)
            def _():
                scatter_wait(k0 - 1)

            compute(0)
            scatter_issue(k0)

            issue_gathers(k0 + 2, 0)
            wait_gathers(k0 + 1, 1)
            scatter_wait(k0)
            compute(1)
            scatter_issue(k0 + 1)
            return carry

        lax.fori_loop(0, (_SEG - 1) // 2, pair_body, 0)

        wait_gathers(_SEG - 1, 0)
        scatter_wait(_SEG - 2)
        compute(0)
        scatter_issue(_SEG - 1)
        scatter_wait(_SEG - 1)

    plsc.subcore_barrier()

    def out_body(k, carry):
        r0 = tile_row0 + k * _RZ
        pltpu.sync_copy(acc_sh.at[pl.ds(r0, _RZ)], zbuf_v)
        pltpu.sync_copy(zbuf_v, out_hbm.at[cid, pl.ds(r0, _RZ)])
        return carry

    lax.fori_loop(0, _RPT // _RZ, out_body, 0)


def _make_edge_call():
    mesh = plsc.VectorSubcoreMesh(
        core_axis_name="c", subcore_axis_name="s",
        num_cores=_NC, num_subcores=_NS)
    return pl.kernel(
        _edge_body,
        out_type=jax.ShapeDtypeStruct((_NC, _NPAD, _PW), jnp.float32),
        mesh=mesh,
        compiler_params=pltpu.CompilerParams(use_tc_tiling_on_sc=False,
                                            needs_layout_passes=False),
        scratch_types=[
            pltpu.VMEM((_SEG, _B), jnp.int32),
            pltpu.VMEM((_SEG, _B), jnp.int32),
            pltpu.VMEM((_B, 16), jnp.float32),
            pltpu.VMEM((_B, 16), jnp.float32),
            pltpu.VMEM((_B, _D), jnp.bfloat16),
            pltpu.VMEM((_B, 16), jnp.float32),
            pltpu.VMEM((_B, 16), jnp.float32),
            pltpu.VMEM((_B, _D), jnp.bfloat16),
            pltpu.VMEM((_B, _PW), jnp.float32),
            pltpu.VMEM((_RZ, _PW), jnp.float32),
            pltpu.VMEM_SHARED((_NPAD, _PW), jnp.float32),
            pltpu.SemaphoreType.DMA,
            pltpu.SemaphoreType.DMA,
            pltpu.SemaphoreType.DMA,
        ],
    )


def _pack_bf16(h):
    return h.astype(jnp.bfloat16)


def _perm_matrix():
    p = jnp.arange(_D)
    g, t = p // 32, p % 32
    ch = 32 * g + 2 * (t % 16) + t // 16
    return (ch[:, None] == jnp.arange(_D)[None, :]).astype(jnp.float32)


def kernel(x, edge_index, W1, as1, ad1, b1, W2, as2, ad2, b2, Wout, bout):
    f32 = jnp.float32
    ei2d = edge_index.reshape(2 * (_E // _B), _B)

    eyeH = jnp.eye(_H, dtype=f32)

    def mk_asd(a_s, a_d):
        As = (a_s[:, :, None] * eyeH[:, None, :]).reshape(_H * _C, _H)
        Ad = (a_d[:, :, None] * eyeH[:, None, :]).reshape(_H * _C, _H)
        return jnp.concatenate([As, Ad], axis=1)

    ASD1 = mk_asd(as1, ad1)
    ASD2 = mk_asd(as2, ad2)
    P8 = jnp.repeat(eyeH, _C, axis=1)
    PERM = _perm_matrix()

    edge_call = _make_edge_call()

    h1, ts1, td1 = _dense(x, W1, ASD1)
    acc1 = edge_call(_pack_bf16(h1), ts1, td1, ei2d)
    h2, ts2, td2 = _combine_mid(acc1, PERM, h1, ts1, td1, b1[None, :], P8,
                                W2, ASD2)
    acc2 = edge_call(_pack_bf16(h2), ts2, td2, ei2d)
    return _final(acc2, PERM, h2, ts2, td2, b2[None, :], P8, Wout,
                  bout[None, :])


# speedup vs baseline: 1.0106x; 1.0106x over previous
import jax
import jax.numpy as jnp
from jax import lax
from jax.experimental import pallas as pl
from jax.experimental.pallas import tpu as pltpu
from jax.experimental.pallas import tpu_sc as plsc

_N = 10000
_E = 320000
_D = 128
_H = 8
_C = 16
_OUT = 64
_PW = 144
_NC = 2
_NS = 16
_NW = _NC * _NS
_EPW = _E // _NW
_B = 80
_CPW = _EPW // _B
_SEG = 25
_NSEG = _CPW // _SEG
_NPAD = 10240
_RPT = _NPAD // _NS
_RZ = 16
_BLK = 2000


def _dense_body(x_ref, w_ref, asd_ref, h_ref, hb_ref, ts_ref, td_ref):
    h = jnp.dot(x_ref[...], w_ref[...], preferred_element_type=jnp.float32)
    h_ref[...] = h
    hb_ref[...] = h.astype(jnp.bfloat16)
    sal = jnp.dot(h, asd_ref[...], preferred_element_type=jnp.float32)
    ts_ref[...] = jnp.concatenate([sal[:, :_H], sal[:, :_H]], axis=1)
    td_ref[...] = jnp.concatenate([sal[:, _H:], sal[:, _H:]], axis=1)


def _dense(x, W, ASD):
    return pl.pallas_call(
        _dense_body,
        grid=(_N // _BLK,),
        in_specs=[
            pl.BlockSpec((_BLK, _D), lambda i: (i, 0)),
            pl.BlockSpec((_D, _D), lambda i: (0, 0)),
            pl.BlockSpec((_D, 2 * _H), lambda i: (0, 0)),
        ],
        out_specs=[
            pl.BlockSpec((_BLK, _D), lambda i: (i, 0)),
            pl.BlockSpec((_BLK, _D), lambda i: (i, 0)),
            pl.BlockSpec((_BLK, 2 * _H), lambda i: (i, 0)),
            pl.BlockSpec((_BLK, 2 * _H), lambda i: (i, 0)),
        ],
        out_shape=[
            jax.ShapeDtypeStruct((_N, _D), jnp.float32),
            jax.ShapeDtypeStruct((_N, _D), jnp.bfloat16),
            jax.ShapeDtypeStruct((_N, 2 * _H), jnp.float32),
            jax.ShapeDtypeStruct((_N, 2 * _H), jnp.float32),
        ],
    )(x, W, ASD)


def _combine_common(acc_ref, perm_ref, h_ref, ts_ref, td_ref, b_ref, p8_ref):
    a0 = acc_ref[0]
    a1 = acc_ref[1]
    nump = a0[:, :_D] + a1[:, :_D]
    num = jnp.dot(nump, perm_ref[...], preferred_element_type=jnp.float32)
    den8 = a0[:, _D:_D + _H] + a1[:, _D:_D + _H]
    es = ts_ref[:, :_H] + td_ref[:, :_H]
    ws8 = jnp.exp(jnp.maximum(es, 0.2 * es))
    p8 = p8_ref[...]
    wsbig = jnp.dot(ws8, p8, preferred_element_type=jnp.float32)
    denbig = jnp.dot(den8 + ws8, p8, preferred_element_type=jnp.float32)
    v = (num + wsbig * h_ref[...]) / denbig + b_ref[...]
    return jnp.where(v > 0, v, jnp.exp(jnp.minimum(v, 0.0)) - 1.0)


def _combine_mid_body(acc_ref, perm_ref, h_ref, ts_ref, td_ref, b_ref, p8_ref,
                      w_ref, asd_ref, h2_ref, hb2_ref, ts2_ref, td2_ref):
    x2 = _combine_common(acc_ref, perm_ref, h_ref, ts_ref, td_ref, b_ref,
                         p8_ref)
    h2 = jnp.dot(x2, w_ref[...], preferred_element_type=jnp.float32)
    h2_ref[...] = h2
    hb2_ref[...] = h2.astype(jnp.bfloat16)
    sal2 = jnp.dot(h2, asd_ref[...], preferred_element_type=jnp.float32)
    ts2_ref[...] = jnp.concatenate([sal2[:, :_H], sal2[:, :_H]], axis=1)
    td2_ref[...] = jnp.concatenate([sal2[:, _H:], sal2[:, _H:]], axis=1)


def _combine_mid(acc, PERM, h, ts, td, b2d, P8, W, ASD):
    return pl.pallas_call(
        _combine_mid_body,
        grid=(_N // _BLK,),
        in_specs=[
            pl.BlockSpec((_NC, _BLK, _PW), lambda i: (0, i, 0)),
            pl.BlockSpec((_D, _D), lambda i: (0, 0)),
            pl.BlockSpec((_BLK, _D), lambda i: (i, 0)),
            pl.BlockSpec((_BLK, 2 * _H), lambda i: (i, 0)),
            pl.BlockSpec((_BLK, 2 * _H), lambda i: (i, 0)),
            pl.BlockSpec((1, _D), lambda i: (0, 0)),
            pl.BlockSpec((_H, _D), lambda i: (0, 0)),
            pl.BlockSpec((_D, _D), lambda i: (0, 0)),
            pl.BlockSpec((_D, 2 * _H), lambda i: (0, 0)),
        ],
        out_specs=[
            pl.BlockSpec((_BLK, _D), lambda i: (i, 0)),
            pl.BlockSpec((_BLK, _D), lambda i: (i, 0)),
            pl.BlockSpec((_BLK, 2 * _H), lambda i: (i, 0)),
            pl.BlockSpec((_BLK, 2 * _H), lambda i: (i, 0)),
        ],
        out_shape=[
            jax.ShapeDtypeStruct((_N, _D), jnp.float32),
            jax.ShapeDtypeStruct((_N, _D), jnp.bfloat16),
            jax.ShapeDtypeStruct((_N, 2 * _H), jnp.float32),
            jax.ShapeDtypeStruct((_N, 2 * _H), jnp.float32),
        ],
    )(acc, PERM, h, ts, td, b2d, P8, W, ASD)


def _final_body(acc_ref, perm_ref, h_ref, ts_ref, td_ref, b_ref, p8_ref,
                wout_ref, bout_ref, out_ref):
    x2 = _combine_common(acc_ref, perm_ref, h_ref, ts_ref, td_ref, b_ref,
                         p8_ref)
    z = jnp.dot(x2, wout_ref[...], preferred_element_type=jnp.float32)
    z = z + bout_ref[...]
    out_ref[...] = 1.0 / (1.0 + jnp.exp(-z))


def _final(acc, PERM, h, ts, td, b2d, P8, Wout, bout2d):
    return pl.pallas_call(
        _final_body,
        grid=(_N // _BLK,),
        in_specs=[
            pl.BlockSpec((_NC, _BLK, _PW), lambda i: (0, i, 0)),
            pl.BlockSpec((_D, _D), lambda i: (0, 0)),
            pl.BlockSpec((_BLK, _D), lambda i: (i, 0)),
            pl.BlockSpec((_BLK, 2 * _H), lambda i: (i, 0)),
            pl.BlockSpec((_BLK, 2 * _H), lambda i: (i, 0)),
            pl.BlockSpec((1, _D), lambda i: (0, 0)),
            pl.BlockSpec((_H, _D), lambda i: (0, 0)),
            pl.BlockSpec((_D, _OUT), lambda i: (0, 0)),
            pl.BlockSpec((1, _OUT), lambda i: (0, 0)),
        ],
        out_specs=pl.BlockSpec((_BLK, _OUT), lambda i: (i, 0)),
        out_shape=jax.ShapeDtypeStruct((_N, _OUT), jnp.float32),
    )(acc, PERM, h, ts, td, b2d, P8, Wout, bout2d)


def _edge_body(hpk_hbm, ts_hbm, td_hbm, ei2d_hbm, out_hbm,
               sseg_v, dseg_v,
               sal_s0, sal_d0, hpk0,
               sal_s1, sal_d1, hpk1,
               pay_v, zbuf_v, acc_sh,
               gsem0, gsem1, ssem):
    cid = lax.axis_index("c")
    sid = lax.axis_index("s")
    wid = sid * _NC + cid
    tile_row0 = sid * _RPT

    bufs = ((sal_s0, sal_d0, hpk0, gsem0),
            (sal_s1, sal_d1, hpk1, gsem1))

    for r in range(_RZ):
        for j in range(_PW // 16):
            zbuf_v[r, pl.ds(j * 16, 16)] = jnp.zeros((16,), jnp.float32)

    def zero_body(k, carry):
        pltpu.sync_copy(zbuf_v, acc_sh.at[pl.ds(tile_row0 + k * _RZ, _RZ)])
        return carry

    lax.fori_loop(0, _RPT // _RZ, zero_body, 0)
    plsc.subcore_barrier()

    iota16 = lax.iota(jnp.int32, 16)
    half = iota16 // 8

    def gather_copies(k, b):
        sal_s, sal_d, hpk, gsem = bufs[b]
        sidx = sseg_v.at[k]
        didx = dseg_v.at[k]
        return (pltpu.make_async_copy(ts_hbm.at[sidx], sal_s, gsem),
                pltpu.make_async_copy(td_hbm.at[didx], sal_d, gsem),
                pltpu.make_async_copy(hpk_hbm.at[sidx], hpk, gsem))

    def issue_gathers(k, b):
        for c in gather_copies(k, b):
            c.start()

    def wait_gathers(k, b):
        for c in gather_copies(k, b):
            c.wait()

    def scatter_issue(k):
        pltpu.async_copy(pay_v, acc_sh.at[dseg_v.at[k]], ssem, add=True)

    def scatter_wait(k):
        pltpu.make_async_copy(pay_v, acc_sh.at[dseg_v.at[k]], ssem).wait()

    def compute(b):
        sal_s, sal_d, hpk, _ = bufs[b]

        @plsc.parallel_loop(0, _B, unroll=2)
        def _(i):
            e = sal_s[i, :] + sal_d[i, :]
            e = jnp.maximum(e, 0.2 * e)
            w = jnp.exp(e)
            pay_v[i, pl.ds(_D, 16)] = w
            for g in range(4):
                v = hpk[i, pl.ds(g * 32, 32)]
                lo, hi = plsc.unpack(v, format=plsc.PackFormat.INTERLEAVED)
                wmix = jnp.take_along_axis(w, 2 * g + half, axis=0)
                pay_v[i, pl.ds(g * 32, 16)] = lo * wmix
                pay_v[i, pl.ds(g * 32 + 16, 16)] = hi * wmix

    for seg in range(_NSEG):
        row0 = wid * _CPW + seg * _SEG
        pltpu.sync_copy(ei2d_hbm.at[pl.ds(row0, _SEG)], sseg_v)
        pltpu.sync_copy(ei2d_hbm.at[pl.ds(_E // _B + row0, _SEG)], dseg_v)

        issue_gathers(0, 0)

        def pair_body(g, carry):
            k0 = 2 * g
            issue_gathers(k0 + 1, 1)
            wait_gathers(k0, 0)

            @pl.when(g > 0)
            def _():
                scatter_wait(k0 - 1)

            compute(0)
            scatter_issue(k0)

            issue_gathers(k0 + 2, 0)
            wait_gathers(k0 + 1, 1)
            scatter_wait(k0)
            compute(1)
            scatter_issue(k0 + 1)
            return carry

        lax.fori_loop(0, (_SEG - 1) // 2, pair_body, 0)

        wait_gathers(_SEG - 1, 0)
        scatter_wait(_SEG - 2)
        compute(0)
        scatter_issue(_SEG - 1)
        scatter_wait(_SEG - 1)

    plsc.subcore_barrier()

    def out_body(k, carry):
        r0 = tile_row0 + k * _RZ
        pltpu.sync_copy(acc_sh.at[pl.ds(r0, _RZ)], zbuf_v)
        pltpu.sync_copy(zbuf_v, out_hbm.at[cid, pl.ds(r0, _RZ)])
        return carry

    lax.fori_loop(0, _RPT // _RZ, out_body, 0)


def _make_edge_call():
    mesh = plsc.VectorSubcoreMesh(
        core_axis_name="c", subcore_axis_name="s",
        num_cores=_NC, num_subcores=_NS)
    return pl.kernel(
        _edge_body,
        out_type=jax.ShapeDtypeStruct((_NC, _NPAD, _PW), jnp.float32),
        mesh=mesh,
        compiler_params=pltpu.CompilerParams(use_tc_tiling_on_sc=False,
                                            needs_layout_passes=False),
        scratch_types=[
            pltpu.VMEM((_SEG, _B), jnp.int32),
            pltpu.VMEM((_SEG, _B), jnp.int32),
            pltpu.VMEM((_B, 16), jnp.float32),
            pltpu.VMEM((_B, 16), jnp.float32),
            pltpu.VMEM((_B, _D), jnp.bfloat16),
            pltpu.VMEM((_B, 16), jnp.float32),
            pltpu.VMEM((_B, 16), jnp.float32),
            pltpu.VMEM((_B, _D), jnp.bfloat16),
            pltpu.VMEM((_B, _PW), jnp.float32),
            pltpu.VMEM((_RZ, _PW), jnp.float32),
            pltpu.VMEM_SHARED((_NPAD, _PW), jnp.float32),
            pltpu.SemaphoreType.DMA,
            pltpu.SemaphoreType.DMA,
            pltpu.SemaphoreType.DMA,
        ],
    )


def _perm_matrix():
    p = jnp.arange(_D)
    g, t = p // 32, p % 32
    ch = 32 * g + 2 * (t % 16) + t // 16
    return (ch[:, None] == jnp.arange(_D)[None, :]).astype(jnp.float32)


def kernel(x, edge_index, W1, as1, ad1, b1, W2, as2, ad2, b2, Wout, bout):
    f32 = jnp.float32
    ei2d = edge_index.reshape(2 * (_E // _B), _B)

    eyeH = jnp.eye(_H, dtype=f32)

    def mk_asd(a_s, a_d):
        As = (a_s[:, :, None] * eyeH[:, None, :]).reshape(_H * _C, _H)
        Ad = (a_d[:, :, None] * eyeH[:, None, :]).reshape(_H * _C, _H)
        return jnp.concatenate([As, Ad], axis=1)

    ASD1 = mk_asd(as1, ad1)
    ASD2 = mk_asd(as2, ad2)
    P8 = jnp.repeat(eyeH, _C, axis=1)
    PERM = _perm_matrix()

    edge_call = _make_edge_call()

    h1, hb1, ts1, td1 = _dense(x, W1, ASD1)
    acc1 = edge_call(hb1, ts1, td1, ei2d)
    h2, hb2, ts2, td2 = _combine_mid(acc1, PERM, h1, ts1, td1, b1[None, :],
                                     P8, W2, ASD2)
    acc2 = edge_call(hb2, ts2, td2, ei2d)
    return _final(acc2, PERM, h2, ts2, td2, b2[None, :], P8, Wout,
                  bout[None, :])


# speedup vs baseline: 1.0866x; 1.0752x over previous
import functools

import jax
import jax.numpy as jnp
from jax import lax
from jax.experimental import pallas as pl
from jax.experimental.pallas import tpu as pltpu
from jax.experimental.pallas import tpu_sc as plsc

_N = 10000
_E = 320000
_D = 128
_H = 8
_C = 16
_OUT = 64
_PW = 144
_NC = 2
_NS = 16
_NW = _NC * _NS
_EPW = _E // _NW
_B = 80
_CPW = _EPW // _B
_SEG = 25
_NSEG = _CPW // _SEG
_NPAD = 10240
_RPT = _NPAD // _NS
_RZ = 32
_BLK = 2000


def _dense_body(x_ref, w_ref, asd_ref, h_ref, ts_ref, td_ref):
    h = jnp.dot(x_ref[...], w_ref[...], preferred_element_type=jnp.float32)
    h_ref[...] = h
    sal = jnp.dot(h, asd_ref[...], preferred_element_type=jnp.float32)
    ts_ref[...] = jnp.concatenate([sal[:, :_H], sal[:, :_H]], axis=1)
    td_ref[...] = jnp.concatenate([sal[:, _H:], sal[:, _H:]], axis=1)


def _dense(x, W, ASD):
    return pl.pallas_call(
        _dense_body,
        grid=(_N // _BLK,),
        in_specs=[
            pl.BlockSpec((_BLK, _D), lambda i: (i, 0)),
            pl.BlockSpec((_D, _D), lambda i: (0, 0)),
            pl.BlockSpec((_D, 2 * _H), lambda i: (0, 0)),
        ],
        out_specs=[
            pl.BlockSpec((_BLK, _D), lambda i: (i, 0)),
            pl.BlockSpec((_BLK, 2 * _H), lambda i: (i, 0)),
            pl.BlockSpec((_BLK, 2 * _H), lambda i: (i, 0)),
        ],
        out_shape=[
            jax.ShapeDtypeStruct((_N, _D), jnp.float32),
            jax.ShapeDtypeStruct((_N, 2 * _H), jnp.float32),
            jax.ShapeDtypeStruct((_N, 2 * _H), jnp.float32),
        ],
    )(x, W, ASD)


def _combine_common(msg_ref, den_ref, h_ref, ts_ref, td_ref, b_ref, p8_ref):
    num = msg_ref[0] + msg_ref[1]
    d0 = den_ref[0]
    d1 = den_ref[1]
    den8 = d0[:, :_H] + d1[:, :_H]
    es = ts_ref[:, :_H] + td_ref[:, :_H]
    ws8 = jnp.exp(jnp.maximum(es, 0.2 * es))
    p8 = p8_ref[...]
    wsbig = jnp.dot(ws8, p8, preferred_element_type=jnp.float32)
    denbig = jnp.dot(den8 + ws8, p8, preferred_element_type=jnp.float32)
    v = (num + wsbig * h_ref[...]) / denbig + b_ref[...]
    return jnp.where(v > 0, v, jnp.exp(jnp.minimum(v, 0.0)) - 1.0)


def _combine_mid_body(msg_ref, den_ref, h_ref, ts_ref, td_ref, b_ref, p8_ref,
                      w_ref, asd_ref, h2_ref, ts2_ref, td2_ref):
    x2 = _combine_common(msg_ref, den_ref, h_ref, ts_ref, td_ref, b_ref,
                         p8_ref)
    h2 = jnp.dot(x2, w_ref[...], preferred_element_type=jnp.float32)
    h2_ref[...] = h2
    sal2 = jnp.dot(h2, asd_ref[...], preferred_element_type=jnp.float32)
    ts2_ref[...] = jnp.concatenate([sal2[:, :_H], sal2[:, :_H]], axis=1)
    td2_ref[...] = jnp.concatenate([sal2[:, _H:], sal2[:, _H:]], axis=1)


def _combine_mid(msg, den, h, ts, td, b2d, P8, W, ASD):
    return pl.pallas_call(
        _combine_mid_body,
        grid=(_N // _BLK,),
        in_specs=[
            pl.BlockSpec((_NC, _BLK, _D), lambda i: (0, i, 0)),
            pl.BlockSpec((_NC, _BLK, 16), lambda i: (0, i, 0)),
            pl.BlockSpec((_BLK, _D), lambda i: (i, 0)),
            pl.BlockSpec((_BLK, 2 * _H), lambda i: (i, 0)),
            pl.BlockSpec((_BLK, 2 * _H), lambda i: (i, 0)),
            pl.BlockSpec((1, _D), lambda i: (0, 0)),
            pl.BlockSpec((_H, _D), lambda i: (0, 0)),
            pl.BlockSpec((_D, _D), lambda i: (0, 0)),
            pl.BlockSpec((_D, 2 * _H), lambda i: (0, 0)),
        ],
        out_specs=[
            pl.BlockSpec((_BLK, _D), lambda i: (i, 0)),
            pl.BlockSpec((_BLK, 2 * _H), lambda i: (i, 0)),
            pl.BlockSpec((_BLK, 2 * _H), lambda i: (i, 0)),
        ],
        out_shape=[
            jax.ShapeDtypeStruct((_N, _D), jnp.float32),
            jax.ShapeDtypeStruct((_N, 2 * _H), jnp.float32),
            jax.ShapeDtypeStruct((_N, 2 * _H), jnp.float32),
        ],
    )(msg, den, h, ts, td, b2d, P8, W, ASD)


def _final_body(msg_ref, den_ref, h_ref, ts_ref, td_ref, b_ref, p8_ref,
                wout_ref, bout_ref, out_ref):
    x2 = _combine_common(msg_ref, den_ref, h_ref, ts_ref, td_ref, b_ref,
                         p8_ref)
    z = jnp.dot(x2, wout_ref[...], preferred_element_type=jnp.float32)
    z = z + bout_ref[...]
    out_ref[...] = 1.0 / (1.0 + jnp.exp(-z))


def _final(msg, den, h, ts, td, b2d, P8, Wout, bout2d):
    return pl.pallas_call(
        _final_body,
        grid=(_N // _BLK,),
        in_specs=[
            pl.BlockSpec((_NC, _BLK, _D), lambda i: (0, i, 0)),
            pl.BlockSpec((_NC, _BLK, 16), lambda i: (0, i, 0)),
            pl.BlockSpec((_BLK, _D), lambda i: (i, 0)),
            pl.BlockSpec((_BLK, 2 * _H), lambda i: (i, 0)),
            pl.BlockSpec((_BLK, 2 * _H), lambda i: (i, 0)),
            pl.BlockSpec((1, _D), lambda i: (0, 0)),
            pl.BlockSpec((_H, _D), lambda i: (0, 0)),
            pl.BlockSpec((_D, _OUT), lambda i: (0, 0)),
            pl.BlockSpec((1, _OUT), lambda i: (0, 0)),
        ],
        out_specs=pl.BlockSpec((_BLK, _OUT), lambda i: (i, 0)),
        out_shape=jax.ShapeDtypeStruct((_N, _OUT), jnp.float32),
    )(msg, den, h, ts, td, b2d, P8, Wout, bout2d)


def _edge_body(h_hbm, ts_hbm, td_hbm, ei2d_hbm, msg_hbm, den_hbm,
               sseg_v, dseg_v,
               sal_s0, sal_d0, pay0, wbuf0,
               sal_s1, sal_d1, pay1, wbuf1,
               zmsg_v, zden_v, accm_sh, accd_sh,
               gsem0, gsem1, ssem0, ssem1):
    cid = lax.axis_index("c")
    sid = lax.axis_index("s")
    wid = sid * _NC + cid
    tile_row0 = sid * _RPT

    bufs = ((sal_s0, sal_d0, pay0, wbuf0, gsem0, ssem0),
            (sal_s1, sal_d1, pay1, wbuf1, gsem1, ssem1))

    for r in range(_RZ):
        for j in range(_D // 16):
            zmsg_v[r, pl.ds(j * 16, 16)] = jnp.zeros((16,), jnp.float32)
        zden_v[r, pl.ds(0, 16)] = jnp.zeros((16,), jnp.float32)

    def zero_body(k, carry):
        pltpu.sync_copy(zmsg_v, accm_sh.at[pl.ds(tile_row0 + k * _RZ, _RZ)])
        pltpu.sync_copy(zden_v, accd_sh.at[pl.ds(tile_row0 + k * _RZ, _RZ)])
        return carry

    lax.fori_loop(0, _RPT // _RZ, zero_body, 0)
    plsc.subcore_barrier()

    def gather_copies(k, b):
        sal_s, sal_d, pay, _, gsem, _ = bufs[b]
        sidx = sseg_v.at[k]
        didx = dseg_v.at[k]
        return (pltpu.make_async_copy(ts_hbm.at[sidx], sal_s, gsem),
                pltpu.make_async_copy(td_hbm.at[didx], sal_d, gsem),
                pltpu.make_async_copy(h_hbm.at[sidx], pay, gsem))

    def issue_gathers(k, b):
        for c in gather_copies(k, b):
            c.start()

    def wait_gathers(k, b):
        for c in gather_copies(k, b):
            c.wait()

    def scatter_copies(k, b):
        _, _, pay, wbuf, _, ssem = bufs[b]
        didx = dseg_v.at[k]
        return (pltpu.make_async_copy(pay, accm_sh.at[didx], ssem),
                pltpu.make_async_copy(wbuf, accd_sh.at[didx], ssem))

    def scatter_issue(k, b):
        _, _, pay, wbuf, _, ssem = bufs[b]
        didx = dseg_v.at[k]
        pltpu.async_copy(pay, accm_sh.at[didx], ssem, add=True)
        pltpu.async_copy(wbuf, accd_sh.at[didx], ssem, add=True)

    def scatter_wait(k, b):
        for c in scatter_copies(k, b):
            c.wait()

    def compute(b):
        sal_s, sal_d, pay, wbuf, _, _ = bufs[b]

        @plsc.parallel_loop(0, _B, unroll=4)
        def _(i):
            e = sal_s[i, :] + sal_d[i, :]
            e = jnp.maximum(e, 0.2 * e)
            w = jnp.exp(e)
            wbuf[i, :] = w
            for j in range(_H):
                wj = jnp.take_along_axis(w, jnp.full((16,), j, jnp.int32),
                                         axis=0)
                pay[i, pl.ds(j * 16, 16)] = pay[i, pl.ds(j * 16, 16)] * wj

    for seg in range(_NSEG):
        row0 = wid * _CPW + seg * _SEG
        pltpu.sync_copy(ei2d_hbm.at[pl.ds(row0, _SEG)], sseg_v)
        pltpu.sync_copy(ei2d_hbm.at[pl.ds(_E // _B + row0, _SEG)], dseg_v)

        issue_gathers(0, 0)

        def pair_body(g, carry):
            k0 = 2 * g
            issue_gathers(k0 + 1, 1)
            wait_gathers(k0, 0)

            @pl.when(g > 0)
            def _():
                scatter_wait(k0 - 2, 0)

            compute(0)
            scatter_issue(k0, 0)

            issue_gathers(k0 + 2, 0)
            wait_gathers(k0 + 1, 1)

            @pl.when(g > 0)
            def _():
                scatter_wait(k0 - 1, 1)

            compute(1)
            scatter_issue(k0 + 1, 1)
            return carry

        lax.fori_loop(0, (_SEG - 1) // 2, pair_body, 0)

        wait_gathers(_SEG - 1, 0)
        scatter_wait(_SEG - 3, 0)
        compute(0)
        scatter_issue(_SEG - 1, 0)
        scatter_wait(_SEG - 2, 1)
        scatter_wait(_SEG - 1, 0)

    plsc.subcore_barrier()

    def out_body(k, carry):
        r0 = tile_row0 + k * _RZ
        pltpu.sync_copy(accm_sh.at[pl.ds(r0, _RZ)], zmsg_v)
        pltpu.sync_copy(zmsg_v, msg_hbm.at[cid, pl.ds(r0, _RZ)])
        pltpu.sync_copy(accd_sh.at[pl.ds(r0, _RZ)], zden_v)
        pltpu.sync_copy(zden_v, den_hbm.at[cid, pl.ds(r0, _RZ)])
        return carry

    lax.fori_loop(0, _RPT // _RZ, out_body, 0)


def _make_edge_call():
    mesh = plsc.VectorSubcoreMesh(
        core_axis_name="c", subcore_axis_name="s",
        num_cores=_NC, num_subcores=_NS)
    return pl.kernel(
        _edge_body,
        out_type=(jax.ShapeDtypeStruct((_NC, _NPAD, _D), jnp.float32),
                  jax.ShapeDtypeStruct((_NC, _NPAD, 16), jnp.float32)),
        mesh=mesh,
        compiler_params=pltpu.CompilerParams(use_tc_tiling_on_sc=False),
        scratch_types=[
            pltpu.VMEM((_SEG, _B), jnp.int32),
            pltpu.VMEM((_SEG, _B), jnp.int32),
            pltpu.VMEM((_B, 16), jnp.float32),
            pltpu.VMEM((_B, 16), jnp.float32),
            pltpu.VMEM((_B, _D), jnp.float32),
            pltpu.VMEM((_B, 16), jnp.float32),
            pltpu.VMEM((_B, 16), jnp.float32),
            pltpu.VMEM((_B, 16), jnp.float32),
            pltpu.VMEM((_B, _D), jnp.float32),
            pltpu.VMEM((_B, 16), jnp.float32),
            pltpu.VMEM((_RZ, _D), jnp.float32),
            pltpu.VMEM((_RZ, 16), jnp.float32),
            pltpu.VMEM_SHARED((_NPAD, _D), jnp.float32),
            pltpu.VMEM_SHARED((_NPAD, 16), jnp.float32),
            pltpu.SemaphoreType.DMA,
            pltpu.SemaphoreType.DMA,
            pltpu.SemaphoreType.DMA,
            pltpu.SemaphoreType.DMA,
        ],
    )


def kernel(x, edge_index, W1, as1, ad1, b1, W2, as2, ad2, b2, Wout, bout):
    f32 = jnp.float32
    ei2d = edge_index.reshape(2 * (_E // _B), _B)

    eyeH = jnp.eye(_H, dtype=f32)

    def mk_asd(a_s, a_d):
        As = (a_s[:, :, None] * eyeH[:, None, :]).reshape(_H * _C, _H)
        Ad = (a_d[:, :, None] * eyeH[:, None, :]).reshape(_H * _C, _H)
        return jnp.concatenate([As, Ad], axis=1)

    ASD1 = mk_asd(as1, ad1)
    ASD2 = mk_asd(as2, ad2)
    P8 = jnp.repeat(eyeH, _C, axis=1)

    edge_call = _make_edge_call()

    h1, ts1, td1 = _dense(x, W1, ASD1)
    msg1, den1 = edge_call(h1, ts1, td1, ei2d)
    h2, ts2, td2 = _combine_mid(msg1, den1, h1, ts1, td1, b1[None, :], P8,
                                W2, ASD2)
    msg2, den2 = edge_call(h2, ts2, td2, ei2d)
    return _final(msg2, den2, h2, ts2, td2, b2[None, :], P8, Wout,
                  bout[None, :])
